# baseline (speedup 1.0000x reference)
import functools

import jax
import jax.numpy as jnp
from jax import lax
from jax.experimental import pallas as pl
from jax.experimental.pallas import tpu as pltpu
from jax.experimental.pallas import tpu_sc as plsc

NUM_NODES = 100000
FEAT_DIM = 128
EMB_DIM = 64
BATCH = 16384
OUT_DIM = FEAT_DIM + EMB_DIM

NC = 2
NS = 16
NW = NC * NS
BPW = BATCH // NW
NCHUNK = 4
C = BPW // NCHUNK
L = 16
G = C // L

TW = 2 * FEAT_DIM
NBLK = NUM_NODES // TW
TAIL = NUM_NODES - NBLK * TW
NPAIR = NUM_NODES // 2
NPPAD = NPAIR + 48
PPW = TW // 2

_mesh = plsc.VectorSubcoreMesh(core_axis_name="c", subcore_axis_name="s")


@functools.partial(
    pl.kernel,
    mesh=_mesh,
    out_type=jax.ShapeDtypeStruct((NPPAD, FEAT_DIM), jnp.float32),
    scratch_types=[
        pltpu.VMEM((2, EMB_DIM, TW), jnp.float32),
        pltpu.VMEM((2, PPW, FEAT_DIM), jnp.float32),
        pltpu.SemaphoreType.DMA,
        pltpu.SemaphoreType.DMA,
    ],
    compiler_params=pltpu.CompilerParams(needs_layout_passes=False),
)
def _table_transpose(et_hbm, tail_hbm, embp_hbm, ebk, tbk, rsem, wsem):
    wid = lax.axis_index("s") * NC + lax.axis_index("c")
    lane = lax.iota(jnp.int32, L)
    rvecs = [lane + (g * L) for g in range(EMB_DIM // L)]

    @pl.when(wid == 0)
    def _():
        pltpu.sync_copy(tail_hbm, embp_hbm.at[pl.ds(NBLK * PPW, TAIL // 2)])

    def nb(i):
        return jnp.minimum(wid + NW * i, NBLK - 1)

    def rd_copy(i, s):
        return pltpu.make_async_copy(
            et_hbm.at[pl.ds(0, EMB_DIM), pl.ds(pl.multiple_of(nb(i) * TW, TW),
                                               TW)],
            ebk.at[s], rsem)

    def wr_copy(i, s):
        return pltpu.make_async_copy(
            tbk.at[s],
            embp_hbm.at[pl.ds(pl.multiple_of(nb(i) * PPW, PPW), PPW)], wsem)

    rd_copy(0, 0).start()

    def step(i, carry):
        s = i % 2
        rd_copy(i, s).wait()

        @pl.when(i + 1 < (NBLK + NW - 1) // NW)
        def _():
            rd_copy(i + 1, 1 - s).start()

        def tr(f, c2, s=s):
            cvec = (lane + f) & (TW - 1)
            prow = lax.shift_right_logical(cvec, 1)
            pbase = (cvec & 1) * EMB_DIM
            for g in range(EMB_DIM // L):
                v = plsc.load_gather(ebk.at[s], [rvecs[g], cvec])
                plsc.store_scatter(tbk.at[s], [prow, pbase + rvecs[g]], v)
            return c2

        lax.fori_loop(0, TW, tr, 0, unroll=2)

        @pl.when(i >= 2)
        def _():
            wr_copy(i - 2, s).wait()

        wr_copy(i, s).start()
        return carry

    nsteps = (NBLK + NW - 1) // NW
    lax.fori_loop(0, nsteps, step, 0)
    wr_copy(nsteps - 2, (nsteps - 2) % 2).wait()
    wr_copy(nsteps - 1, (nsteps - 1) % 2).wait()


@functools.partial(
    pl.kernel,
    mesh=_mesh,
    out_type=jax.ShapeDtypeStruct((OUT_DIM, BATCH), jnp.float32),
    scratch_types=[
        pltpu.VMEM((BPW,), jnp.int32),
        pltpu.VMEM((BPW,), jnp.int32),
        pltpu.VMEM((2, C, FEAT_DIM), jnp.float32),
        pltpu.VMEM((2, C, FEAT_DIM), jnp.float32),
        pltpu.VMEM((2, FEAT_DIM, C), jnp.float32),
        pltpu.VMEM((2, EMB_DIM, C), jnp.float32),
        pltpu.SemaphoreType.DMA,
        pltpu.SemaphoreType.DMA,
        pltpu.SemaphoreType.DMA,
    ],
    compiler_params=pltpu.CompilerParams(needs_layout_passes=False),
)
def _encoder(idx_hbm, feat_hbm, embp_hbm, out_hbm, idx_v, ix2_v, fbuf, ebuf,
             tf, te, gsem0, gsem1, wsem):
    wid = lax.axis_index("s") * NC + lax.axis_index("c")
    base = wid * BPW
    pltpu.sync_copy(idx_hbm.at[pl.ds(base, BPW)], idx_v)
    lane = lax.iota(jnp.int32, L)

    def pair_ix(i, carry):
        v = idx_v[pl.ds(i * L, L)]
        ix2_v[pl.ds(i * L, L)] = lax.shift_right_logical(v, 1)
        return carry

    lax.fori_loop(0, BPW // L, pair_ix, 0, unroll=4)

    gsems = (gsem0, gsem1)

    def start_gathers(j):
        s = j % 2
        cf = pltpu.async_copy(
            feat_hbm.at[idx_v.at[pl.ds(j * C, C)]], fbuf.at[s], gsems[s])
        ce = pltpu.async_copy(
            embp_hbm.at[ix2_v.at[pl.ds(j * C, C)]], ebuf.at[s], gsems[s])
        return cf, ce

    pending = start_gathers(0)
    writes = []
    rvecs = [lane + (g * L) for g in range(G)]
    for j in range(NCHUNK):
        s = j % 2
        cf, ce = pending
        cf.wait()
        ce.wait()
        if j + 1 < NCHUNK:
            pending = start_gathers(j + 1)

        def tr_feat(f, carry, s=s):
            cvec = (lane + f) & (FEAT_DIM - 1)
            for g in range(G):
                v = plsc.load_gather(fbuf.at[s], [rvecs[g], cvec])
                plsc.store_scatter(tf.at[s], [cvec, rvecs[g]], v)
            return carry

        lax.fori_loop(0, FEAT_DIM, tr_feat, 0, unroll=2)

        bvecs = []
        for g in range(G):
            ivec = idx_v[pl.ds(j * C + g * L, L)]
            bvecs.append((ivec & 1) * EMB_DIM)

        def tr_emb(f, carry, s=s, bvecs=bvecs):
            cvec = (lane + f) & (EMB_DIM - 1)
            for g in range(G):
                v = plsc.load_gather(ebuf.at[s], [rvecs[g], bvecs[g] + cvec])
                plsc.store_scatter(te.at[s], [cvec, rvecs[g]], v)
            return carry

        lax.fori_loop(0, EMB_DIM, tr_emb, 0, unroll=2)

        if j >= 2:
            for w in writes[j - 2]:
                w.wait()
        col = base + j * C
        wf = pltpu.async_copy(
            tf.at[s], out_hbm.at[pl.ds(0, FEAT_DIM), pl.ds(col, C)], wsem)
        we = pltpu.async_copy(
            te.at[s], out_hbm.at[pl.ds(FEAT_DIM, EMB_DIM), pl.ds(col, C)],
            wsem)
        writes.append((wf, we))

    for pair in writes[-2:]:
        for w in pair:
            w.wait()


def kernel(indices, features, emb_table):
    idx = indices.astype(jnp.int32)
    et = emb_table.T
    tail = emb_table[NBLK * TW:, :].reshape(TAIL // 2, FEAT_DIM)
    embp = _table_transpose(et, tail)
    out_t = _encoder(idx, features, embp)
    return out_t.T


# speedup vs baseline: 1.0908x; 1.0908x over previous
import functools

import jax
import jax.numpy as jnp
from jax import lax
from jax.experimental import pallas as pl
from jax.experimental.pallas import tpu as pltpu
from jax.experimental.pallas import tpu_sc as plsc

NUM_NODES = 100000
FEAT_DIM = 128
EMB_DIM = 64
BATCH = 16384
OUT_DIM = FEAT_DIM + EMB_DIM

NC = 2
NS = 16
NW = NC * NS
BPW = BATCH // NW
NCHUNK = 4
C = BPW // NCHUNK
L = 16
G = C // L

_mesh = plsc.VectorSubcoreMesh(core_axis_name="c", subcore_axis_name="s")


@functools.partial(
    pl.kernel,
    mesh=_mesh,
    out_type=jax.ShapeDtypeStruct((OUT_DIM, BATCH), jnp.float32),
    scratch_types=[
        pltpu.VMEM((BPW,), jnp.int32),
        pltpu.VMEM((2, C, FEAT_DIM), jnp.float32),
        pltpu.VMEM((2, C, FEAT_DIM), jnp.float32),
        pltpu.VMEM((2, FEAT_DIM, C), jnp.float32),
        pltpu.VMEM((2, EMB_DIM, C), jnp.float32),
        pltpu.SemaphoreType.DMA,
        pltpu.SemaphoreType.DMA,
        pltpu.SemaphoreType.DMA,
    ],
    compiler_params=pltpu.CompilerParams(needs_layout_passes=False),
)
def _encoder(idx_hbm, feat_hbm, embp_hbm, out_hbm, idx_v, fbuf, ebuf,
             tf, te, gsem0, gsem1, wsem):
    wid = lax.axis_index("s") * NC + lax.axis_index("c")
    base = wid * BPW
    pltpu.sync_copy(idx_hbm.at[pl.ds(base, BPW)], idx_v)

    gsems = (gsem0, gsem1)

    def start_gathers(j):
        s = j % 2
        ix = idx_v.at[pl.ds(j * C, C)]
        cf = pltpu.async_copy(feat_hbm.at[ix], fbuf.at[s], gsems[s])
        ce = pltpu.async_copy(embp_hbm.at[ix], ebuf.at[s], gsems[s])
        return cf, ce

    pending = start_gathers(0)
    writes = []
    lane = lax.iota(jnp.int32, L)
    for j in range(NCHUNK):
        s = j % 2
        cf, ce = pending
        cf.wait()
        ce.wait()
        if j + 1 < NCHUNK:
            pending = start_gathers(j + 1)

        rvecs = [lane + (g * L) for g in range(G)]

        def tr_feat(f, carry, s=s):
            cvec = (lane + f) & (FEAT_DIM - 1)
            for g in range(G):
                v = plsc.load_gather(fbuf.at[s], [rvecs[g], cvec])
                plsc.store_scatter(tf.at[s], [cvec, rvecs[g]], v)
            return carry

        lax.fori_loop(0, FEAT_DIM, tr_feat, 0, unroll=4)

        def tr_emb(f, carry, s=s):
            cvec = (lane + f) & (EMB_DIM - 1)
            for g in range(G):
                v = plsc.load_gather(ebuf.at[s], [rvecs[g], cvec])
                plsc.store_scatter(te.at[s], [cvec, rvecs[g]], v)
            return carry

        lax.fori_loop(0, EMB_DIM, tr_emb, 0, unroll=4)

        if j >= 2:
            for w in writes[j - 2]:
                w.wait()
        col = base + j * C
        wf = pltpu.async_copy(
            tf.at[s], out_hbm.at[pl.ds(0, FEAT_DIM), pl.ds(col, C)], wsem)
        we = pltpu.async_copy(
            te.at[s], out_hbm.at[pl.ds(FEAT_DIM, EMB_DIM), pl.ds(col, C)],
            wsem)
        writes.append((wf, we))

    for pair in writes[-2:]:
        for w in pair:
            w.wait()


def kernel(indices, features, emb_table):
    idx = indices.astype(jnp.int32)
    emb_p = jnp.pad(emb_table, ((0, 0), (0, FEAT_DIM - EMB_DIM)))
    out_t = _encoder(idx, features, emb_p)
    return out_t.T


# speedup vs baseline: 1.1153x; 1.0225x over previous
import functools

import jax
import jax.numpy as jnp
from jax import lax
from jax.experimental import pallas as pl
from jax.experimental.pallas import tpu as pltpu
from jax.experimental.pallas import tpu_sc as plsc

NUM_NODES = 100000
FEAT_DIM = 128
EMB_DIM = 64
BATCH = 16384
OUT_DIM = FEAT_DIM + EMB_DIM

NC = 2
NS = 16
NW = NC * NS
BPW = BATCH // NW
NCHUNK = 4
C = BPW // NCHUNK
L = 16
G = C // L

_mesh = plsc.VectorSubcoreMesh(core_axis_name="c", subcore_axis_name="s")


@functools.partial(
    pl.kernel,
    mesh=_mesh,
    out_type=jax.ShapeDtypeStruct((OUT_DIM, BATCH), jnp.float32),
    scratch_types=[
        pltpu.VMEM((BPW,), jnp.int32),
        pltpu.VMEM((2, C, FEAT_DIM), jnp.float32),
        pltpu.VMEM((2, C, FEAT_DIM), jnp.float32),
        pltpu.VMEM((2, FEAT_DIM, C), jnp.float32),
        pltpu.VMEM((2, EMB_DIM, C), jnp.float32),
        pltpu.SemaphoreType.DMA,
        pltpu.SemaphoreType.DMA,
        pltpu.SemaphoreType.DMA,
    ],
    compiler_params=pltpu.CompilerParams(needs_layout_passes=False),
)
def _encoder(idx_hbm, feat_hbm, embp_hbm, out_hbm, idx_v, fbuf, ebuf,
             tf, te, gsem0, gsem1, wsem):
    wid = lax.axis_index("s") * NC + lax.axis_index("c")
    base = wid * BPW
    pltpu.sync_copy(idx_hbm.at[pl.ds(base, BPW)], idx_v)

    gsems = (gsem0, gsem1)

    def start_gathers(j):
        s = j % 2
        ix = idx_v.at[pl.ds(j * C, C)]
        cf = pltpu.async_copy(feat_hbm.at[ix], fbuf.at[s], gsems[s])
        ce = pltpu.async_copy(embp_hbm.at[ix], ebuf.at[s], gsems[s])
        return cf, ce

    pending = start_gathers(0)
    writes = []
    lane = lax.iota(jnp.int32, L)
    for j in range(NCHUNK):
        s = j % 2
        cf, ce = pending
        cf.wait()
        ce.wait()
        if j + 1 < NCHUNK:
            pending = start_gathers(j + 1)

        rvecs = [lane + (g * L) for g in range(G)]

        def tr_feat(f, carry, s=s):
            cvec = (lane + f) & (FEAT_DIM - 1)
            for g in range(G):
                v = plsc.load_gather(fbuf.at[s], [rvecs[g], cvec])
                plsc.store_scatter(tf.at[s], [cvec, rvecs[g]], v)
            return carry

        lax.fori_loop(0, FEAT_DIM, tr_feat, 0, unroll=2)

        def tr_emb(f, carry, s=s):
            cvec = (lane + f) & (EMB_DIM - 1)
            for g in range(G):
                v = plsc.load_gather(ebuf.at[s], [rvecs[g], cvec])
                plsc.store_scatter(te.at[s], [cvec, rvecs[g]], v)
            return carry

        lax.fori_loop(0, EMB_DIM, tr_emb, 0, unroll=2)

        if j >= 2:
            for w in writes[j - 2]:
                w.wait()
        col = base + j * C
        wf = pltpu.async_copy(
            tf.at[s], out_hbm.at[pl.ds(0, FEAT_DIM), pl.ds(col, C)], wsem)
        we = pltpu.async_copy(
            te.at[s], out_hbm.at[pl.ds(FEAT_DIM, EMB_DIM), pl.ds(col, C)],
            wsem)
        writes.append((wf, we))

    for pair in writes[-2:]:
        for w in pair:
            w.wait()


def kernel(indices, features, emb_table):
    idx = indices.astype(jnp.int32)
    emb_p = jnp.pad(emb_table, ((0, 0), (0, FEAT_DIM - EMB_DIM)))
    out_t = _encoder(idx, features, emb_p)
    return out_t.T
